# Initial kernel scaffold; baseline (speedup 1.0000x reference)
import jax
import jax.numpy as jnp
from jax.experimental import pallas as pl


def kernel(hidden_states, gate_weight, gate_up_proj, gate_up_bias, down_proj, down_bias):
    raise NotImplementedError("write your pallas kernel here")


# speedup vs baseline: 1.1060x; 1.1060x over previous
import jax
import jax.numpy as jnp
from jax.experimental import pallas as pl
from jax.experimental.pallas import tpu as pltpu

_E = 8
_K = 2
_D = 768
_F = 2048
_T = 2048
_B = 256
_A = _T * _K
_NB = _A // _B + (_E - 1)
_PAD = _NB * _B
_EPAD = 128


def _router_kernel(x_ref, gwt_ref, logits_ref, w2_ref, e2_ref):
    x = x_ref[...]
    logits = jnp.dot(x, gwt_ref[...], preferred_element_type=jnp.float32)
    logits_ref[...] = logits
    col = jax.lax.broadcasted_iota(jnp.int32, (_T, _EPAD), 1)
    valid = col < _E
    masked = jnp.where(valid, logits, -jnp.inf)
    m = jnp.max(masked, axis=1, keepdims=True)
    p = jnp.where(valid, jnp.exp(masked - m), 0.0)
    probs = p / jnp.sum(p, axis=1, keepdims=True)
    m0 = jnp.max(probs, axis=1, keepdims=True)
    e0 = jnp.min(jnp.where((probs == m0) & valid, col, _EPAD), axis=1,
                 keepdims=True)
    probs1 = jnp.where(col == e0, -1.0, probs)
    m1 = jnp.max(probs1, axis=1, keepdims=True)
    e1 = jnp.min(jnp.where((probs1 == m1) & valid, col, _EPAD), axis=1,
                 keepdims=True)
    s = m0 + m1
    w2_ref[...] = jnp.where(col == 0, m0 / s, jnp.where(col == 1, m1 / s, 0.0))
    e2_ref[...] = jnp.where(col == 0, e0, jnp.where(col == 1, e1, 0))


def _ffn_kernel(be_ref, act_ref, tid_ref,
                x_ref, g_ref, u_ref, gb_ref, ub_ref, dp_ref, db_ref, w_ref,
                out_ref, xs_ref):
    b = pl.program_id(0)

    @pl.when(b == 0)
    def _init():
        out_ref[...] = jnp.zeros_like(out_ref)

    @pl.when(act_ref[b] == 1)
    def _work():
        base = b * _B

        def gather(i, carry):
            t = tid_ref[base + i]
            xs_ref[pl.ds(i, 1), :] = x_ref[pl.ds(t, 1), :]
            return carry

        jax.lax.fori_loop(0, _B, gather, 0)
        xs = xs_ref[...]
        gate = jnp.dot(xs, g_ref[0], preferred_element_type=jnp.float32) \
            + gb_ref[0]
        up = jnp.dot(xs, u_ref[0], preferred_element_type=jnp.float32) \
            + ub_ref[0]
        inter = gate * jax.lax.logistic(gate) * up
        o = jnp.dot(inter, dp_ref[0], preferred_element_type=jnp.float32) \
            + db_ref[0]
        xs_ref[...] = o * w_ref[...]

        def scatter(i, carry):
            t = tid_ref[base + i]
            out_ref[pl.ds(t, 1), :] = (
                out_ref[pl.ds(t, 1), :] + xs_ref[pl.ds(i, 1), :])
            return carry

        jax.lax.fori_loop(0, _B, scatter, 0)


def kernel(hidden_states, gate_weight, gate_up_proj, gate_up_bias,
           down_proj, down_bias):
    x = hidden_states
    gwt = jnp.zeros((_D, _EPAD), jnp.float32).at[:, :_E].set(gate_weight.T)
    logits_pad, w2, e2 = pl.pallas_call(
        _router_kernel,
        out_shape=[
            jax.ShapeDtypeStruct((_T, _EPAD), jnp.float32),
            jax.ShapeDtypeStruct((_T, _EPAD), jnp.float32),
            jax.ShapeDtypeStruct((_T, _EPAD), jnp.int32),
        ],
    )(x, gwt)
    router_logits = logits_pad[:, :_E]

    flat_e = e2[:, :_K].reshape(-1)
    flat_w = w2[:, :_K].reshape(-1)
    counts = jnp.bincount(flat_e, length=_E)
    blocks_per_e = (counts + _B - 1) // _B
    cum_blocks = jnp.cumsum(blocks_per_e)
    block_start = (cum_blocks - blocks_per_e) * _B
    cc = jnp.cumsum(counts) - counts
    order = jnp.argsort(flat_e)
    se = flat_e[order]
    dest = block_start[se] + (jnp.arange(_A, dtype=jnp.int32) - cc[se])
    tid = jnp.zeros((_PAD,), jnp.int32).at[dest].set(
        (order // _K).astype(jnp.int32))
    wpad = jnp.zeros((_PAD, 1), jnp.float32).at[dest, 0].set(flat_w[order])
    total_blocks = cum_blocks[-1]
    bidx = jnp.arange(_NB, dtype=jnp.int32)
    raw = jnp.searchsorted(cum_blocks, bidx, side='right')
    last_e = jnp.searchsorted(cum_blocks, total_blocks - 1, side='right')
    block_expert = jnp.where(bidx < total_blocks, raw, last_e).astype(jnp.int32)
    block_active = (bidx < total_blocks).astype(jnp.int32)

    gproj = gate_up_proj[:, :, :_F]
    uproj = gate_up_proj[:, :, _F:]
    gb = gate_up_bias[:, :_F].reshape(_E, 1, _F)
    ub = gate_up_bias[:, _F:].reshape(_E, 1, _F)
    dbias = down_bias.reshape(_E, 1, _D)

    grid_spec = pltpu.PrefetchScalarGridSpec(
        num_scalar_prefetch=3,
        grid=(_NB,),
        in_specs=[
            pl.BlockSpec((_T, _D), lambda b, be, act, tid: (0, 0)),
            pl.BlockSpec((1, _D, _F), lambda b, be, act, tid: (be[b], 0, 0)),
            pl.BlockSpec((1, _D, _F), lambda b, be, act, tid: (be[b], 0, 0)),
            pl.BlockSpec((1, 1, _F), lambda b, be, act, tid: (be[b], 0, 0)),
            pl.BlockSpec((1, 1, _F), lambda b, be, act, tid: (be[b], 0, 0)),
            pl.BlockSpec((1, _F, _D), lambda b, be, act, tid: (be[b], 0, 0)),
            pl.BlockSpec((1, 1, _D), lambda b, be, act, tid: (be[b], 0, 0)),
            pl.BlockSpec((_B, 1), lambda b, be, act, tid: (b, 0)),
        ],
        out_specs=pl.BlockSpec((_T, _D), lambda b, be, act, tid: (0, 0)),
        scratch_shapes=[pltpu.VMEM((_B, _D), jnp.float32)],
    )
    out = pl.pallas_call(
        _ffn_kernel,
        grid_spec=grid_spec,
        out_shape=jax.ShapeDtypeStruct((_T, _D), jnp.float32),
    )(block_expert, block_active, tid, x, gproj, uproj, gb, ub,
      down_proj, dbias, wpad)
    return out, router_logits
